# Initial kernel scaffold; baseline (speedup 1.0000x reference)
import jax
import jax.numpy as jnp
from jax.experimental import pallas as pl


def kernel(logits, noise):
    raise NotImplementedError("write your pallas kernel here")


# speedup vs baseline: 1.4463x; 1.4463x over previous
import functools

import jax
import jax.numpy as jnp
from jax.experimental import pallas as pl
from jax.experimental.pallas import tpu as pltpu

_B, _V = 64, 100000
_ROWS = 8


def _body(lg_ref, nz_ref, probs_ref, ix_ref):
    l = lg_ref[...]
    m = jnp.max(l, axis=-1, keepdims=True)
    e = jnp.exp(l - m)
    s = jnp.sum(e, axis=-1, keepdims=True)
    p = e / s
    probs_ref[...] = p
    g = -jnp.log(-jnp.log(nz_ref[...] + 1e-10) + 1e-10)
    score = jnp.log(p + 1e-10) + g
    mx = jnp.max(score, axis=-1, keepdims=True)
    col = jax.lax.broadcasted_iota(jnp.int32, score.shape, 1)
    idx = jnp.min(jnp.where(score == mx, col, _V), axis=-1)
    ix_ref[...] = idx.astype(jnp.int32)[:, None]


@jax.jit
def kernel(logits, noise):
    grid = (_B // _ROWS,)
    probs, ix = pl.pallas_call(
        _body,
        grid=grid,
        in_specs=[
            pl.BlockSpec((_ROWS, _V), lambda i: (i, 0)),
            pl.BlockSpec((_ROWS, _V), lambda i: (i, 0)),
        ],
        out_specs=[
            pl.BlockSpec((_ROWS, _V), lambda i: (i, 0)),
            pl.BlockSpec((_ROWS, 1), lambda i: (i, 0)),
        ],
        out_shape=[
            jax.ShapeDtypeStruct((_B, _V), jnp.float32),
            jax.ShapeDtypeStruct((_B, 1), jnp.int32),
        ],
        compiler_params=pltpu.CompilerParams(
            dimension_semantics=("arbitrary",),
        ),
    )(logits, noise)
    return probs, ix


# speedup vs baseline: 1.6535x; 1.1432x over previous
import functools

import jax
import jax.numpy as jnp
from jax.experimental import pallas as pl
from jax.experimental.pallas import tpu as pltpu

_B, _V = 64, 100000
_ROWS = 8


def _body(lg_ref, nz_ref, probs_ref, ix_ref):
    l = lg_ref[...]
    m = jnp.max(l, axis=-1, keepdims=True)
    e = jnp.exp(l - m)
    s = jnp.sum(e, axis=-1, keepdims=True)
    p = e * (1.0 / s)
    probs_ref[...] = p
    b = -jnp.log(nz_ref[...] + 1e-10) + 1e-10
    score = (p + 1e-10) / b
    mx = jnp.max(score, axis=-1, keepdims=True)
    col = jax.lax.broadcasted_iota(jnp.int32, score.shape, 1)
    idx = jnp.min(jnp.where(score == mx, col, _V), axis=-1)
    ix_ref[...] = idx.astype(jnp.int32)[:, None]


@jax.jit
def kernel(logits, noise):
    grid = (_B // _ROWS,)
    probs, ix = pl.pallas_call(
        _body,
        grid=grid,
        in_specs=[
            pl.BlockSpec((_ROWS, _V), lambda i: (i, 0)),
            pl.BlockSpec((_ROWS, _V), lambda i: (i, 0)),
        ],
        out_specs=[
            pl.BlockSpec((_ROWS, _V), lambda i: (i, 0)),
            pl.BlockSpec((_ROWS, 1), lambda i: (i, 0)),
        ],
        out_shape=[
            jax.ShapeDtypeStruct((_B, _V), jnp.float32),
            jax.ShapeDtypeStruct((_B, 1), jnp.int32),
        ],
        compiler_params=pltpu.CompilerParams(
            dimension_semantics=("arbitrary",),
        ),
    )(logits, noise)
    return probs, ix
